# baseline (speedup 1.0000x reference)
import jax
import jax.numpy as jnp
from jax import lax
from jax.experimental import pallas as pl
from jax.experimental.pallas import tpu as pltpu
from jax.experimental.pallas import tpu_sc as plsc

B = 16384
N_CONT = 26
D_MODEL = 512
D_UPC = 32
D_STORE = 16

_NC = 2
_NS = 16
_NW = _NC * _NS
_BPW = B // _NW

_TILE = 512
_NB = B // _TILE


def _sc_gather_body(upc_ids, store_ids, upc_table, store_table,
                    upc_out, store_out,
                    uidx_v, sidx_v, urows_v, srows_v, sem_u, sem_s):
    wid = lax.axis_index("s") * _NC + lax.axis_index("c")
    base = wid * _BPW
    pltpu.sync_copy(upc_ids.at[pl.ds(base, _BPW)], uidx_v)
    pltpu.sync_copy(store_ids.at[pl.ds(base, _BPW)], sidx_v)
    cu = pltpu.async_copy(upc_table.at[uidx_v], urows_v, sem_u)
    cs = pltpu.async_copy(store_table.at[sidx_v], srows_v, sem_s)
    cu.wait()
    cs.wait()
    pltpu.sync_copy(urows_v, upc_out.at[pl.ds(base, _BPW)])
    pltpu.sync_copy(srows_v, store_out.at[pl.ds(base, _BPW)])


def _sc_gather(upc_ids, store_ids, upc_table, store_table):
    mesh = plsc.VectorSubcoreMesh(core_axis_name="c", subcore_axis_name="s")
    fn = pl.kernel(
        _sc_gather_body,
        mesh=mesh,
        compiler_params=pltpu.CompilerParams(use_tc_tiling_on_sc=False),
        out_type=[
            jax.ShapeDtypeStruct((B, D_UPC), jnp.bfloat16),
            jax.ShapeDtypeStruct((B, D_STORE), jnp.bfloat16),
        ],
        scratch_types=[
            pltpu.VMEM((_BPW,), jnp.int32),
            pltpu.VMEM((_BPW,), jnp.int32),
            pltpu.VMEM((_BPW, D_UPC), jnp.bfloat16),
            pltpu.VMEM((_BPW, D_STORE), jnp.bfloat16),
            pltpu.SemaphoreType.DMA,
            pltpu.SemaphoreType.DMA,
        ],
    )
    return fn(upc_ids, store_ids, upc_table, store_table)


def _tc_body(mids_ref, cont_ref, upc_ref, store_ref, mtab_ref,
             W1_ref, b1_ref, g1_ref, W2a_ref, W2b_ref, W2m_ref, W2d_ref,
             b2_ref, g2_ref, W3_ref, b3_ref, out_ref):
    eps = jnp.finfo(jnp.float32).eps

    x = cont_ref[...]
    x = jnp.sign(x) * jnp.log1p(jnp.abs(x))
    c = jnp.dot(x, W1_ref[...], preferred_element_type=jnp.float32) + b1_ref[...]
    c = c * jax.nn.sigmoid(c)
    c = c * lax.rsqrt(jnp.mean(c * c, axis=-1, keepdims=True) + eps) * g1_ref[...]

    mids = mids_ref[0, 0, :]
    onehot = (mids[:, None] == lax.broadcasted_iota(jnp.int32, (_TILE, 16), 1))
    memb = jnp.dot(onehot.astype(jnp.float32), mtab_ref[...],
                   preferred_element_type=jnp.float32)

    gu = upc_ref[...].astype(jnp.float32)
    gs = store_ref[...].astype(jnp.float32)
    h = (jnp.dot(gu, W2a_ref[...], preferred_element_type=jnp.float32)
         + jnp.dot(gs, W2b_ref[...], preferred_element_type=jnp.float32)
         + jnp.dot(memb, W2m_ref[...], preferred_element_type=jnp.float32)
         + jnp.dot(c, W2d_ref[...], preferred_element_type=jnp.float32)
         + b2_ref[...])
    h = h * jax.nn.sigmoid(h)
    h = h * lax.rsqrt(jnp.mean(h * h, axis=-1, keepdims=True) + eps) * g2_ref[...]

    out_ref[...] = (jnp.dot(h, W3_ref[...], preferred_element_type=jnp.float32)
                    + b3_ref[...])


def _full(shape):
    return pl.BlockSpec(shape, lambda i: (0,) * len(shape))


def _tc_mlp(month_ids3, continuous_feats, upc_g, store_g, mtab_pad,
            W1, b1, g1, W2a, W2b, W2m, W2d, b2, g2, W3, b3):
    return pl.pallas_call(
        _tc_body,
        grid=(_NB,),
        in_specs=[
            pl.BlockSpec((1, 1, _TILE), lambda i: (i, 0, 0)),
            pl.BlockSpec((_TILE, N_CONT), lambda i: (i, 0)),
            pl.BlockSpec((_TILE, D_UPC), lambda i: (i, 0)),
            pl.BlockSpec((_TILE, D_STORE), lambda i: (i, 0)),
            _full((16, 6)),
            _full((N_CONT, 32)),
            _full((1, 32)),
            _full((1, 32)),
            _full((D_UPC, 128)),
            _full((D_STORE, 128)),
            _full((6, 128)),
            _full((32, 128)),
            _full((1, 128)),
            _full((1, 128)),
            _full((128, D_MODEL)),
            _full((1, D_MODEL)),
        ],
        out_specs=pl.BlockSpec((_TILE, D_MODEL), lambda i: (i, 0)),
        out_shape=jax.ShapeDtypeStruct((B, D_MODEL), jnp.float32),
    )(month_ids3, continuous_feats, upc_g, store_g, mtab_pad,
      W1, b1, g1, W2a, W2b, W2m, W2d, b2, g2, W3, b3)


def kernel(upc_ids, store_ids, continuous_feats, month_ids,
           upc_table, store_table, month_table,
           W1, b1, g1, W2, b2, g2, W3, b3):
    upc_ids = upc_ids.astype(jnp.int32)
    store_ids = store_ids.astype(jnp.int32)
    month_ids3 = month_ids.astype(jnp.int32).reshape(_NB, 1, _TILE)

    upc_g, store_g = _sc_gather(upc_ids, store_ids,
                                upc_table.astype(jnp.bfloat16),
                                store_table.astype(jnp.bfloat16))

    W2a = W2[0:32]
    W2b = W2[32:48]
    W2m = W2[64:70]
    W2d = W2[70:102]
    mtab_pad = jnp.zeros((16, 6), jnp.float32).at[:12].set(month_table)

    return _tc_mlp(month_ids3, continuous_feats, upc_g, store_g, mtab_pad,
                   W1, b1.reshape(1, -1), g1.reshape(1, -1),
                   W2a, W2b, W2m, W2d, b2.reshape(1, -1), g2.reshape(1, -1),
                   W3, b3.reshape(1, -1))


# speedup vs baseline: 1.1857x; 1.1857x over previous
import functools

import jax
import jax.numpy as jnp
from jax import lax
from jax.experimental import pallas as pl
from jax.experimental.pallas import tpu as pltpu
from jax.experimental.pallas import tpu_sc as plsc

B = 16384
N_CONT = 26
D_MODEL = 512
D_UPC = 32
D_STORE = 16

_NC = 2
_NS = 16
_NW = _NC * _NS
_BPW = B // _NW

_TILE = 512
_NB = B // _TILE


def _sc_gather_body(upc_ids, store_ids, upc_table, store_table,
                    upc_out, store_out,
                    uidx_v, sidx_v, urows_v, srows_v, sem_u, sem_s):
    wid = lax.axis_index("s") * _NC + lax.axis_index("c")
    base = wid * _BPW
    pltpu.sync_copy(upc_ids.at[pl.ds(base, _BPW)], uidx_v)
    pltpu.sync_copy(store_ids.at[pl.ds(base, _BPW)], sidx_v)
    cu = pltpu.async_copy(upc_table.at[uidx_v], urows_v, sem_u)
    cs = pltpu.async_copy(store_table.at[sidx_v], srows_v, sem_s)
    cu.wait()
    cs.wait()
    pltpu.sync_copy(urows_v, upc_out.at[pl.ds(base, _BPW)])
    pltpu.sync_copy(srows_v, store_out.at[pl.ds(base, _BPW)])


def _sc_gather(upc_ids, store_ids, upc_table, store_table):
    mesh = plsc.VectorSubcoreMesh(core_axis_name="c", subcore_axis_name="s")
    fn = pl.kernel(
        _sc_gather_body,
        mesh=mesh,
        compiler_params=pltpu.CompilerParams(use_tc_tiling_on_sc=False),
        out_type=[
            jax.ShapeDtypeStruct((B, D_UPC), jnp.float32),
            jax.ShapeDtypeStruct((B, D_STORE), jnp.float32),
        ],
        scratch_types=[
            pltpu.VMEM((_BPW,), jnp.int32),
            pltpu.VMEM((_BPW,), jnp.int32),
            pltpu.VMEM((_BPW, D_UPC), jnp.float32),
            pltpu.VMEM((_BPW, D_STORE), jnp.float32),
            pltpu.SemaphoreType.DMA,
            pltpu.SemaphoreType.DMA,
        ],
    )
    return fn(upc_ids, store_ids, upc_table, store_table)


def _tc_body(mids_ref, cont_ref, upc_ref, store_ref, mtab_ref,
             W1_ref, b1_ref, g1_ref, W2a_ref, W2b_ref, W2m_ref, W2d_ref,
             b2_ref, g2_ref, W3_ref, b3_ref, out_ref):
    eps = jnp.finfo(jnp.float32).eps

    x = cont_ref[...]
    x = jnp.sign(x) * jnp.log1p(jnp.abs(x))
    c = jnp.dot(x, W1_ref[...], preferred_element_type=jnp.float32) + b1_ref[...]
    c = c * jax.nn.sigmoid(c)
    c = c * lax.rsqrt(jnp.mean(c * c, axis=-1, keepdims=True) + eps) * g1_ref[...]

    mids = mids_ref[0, 0, :]
    onehot = (mids[:, None] == lax.broadcasted_iota(jnp.int32, (_TILE, 16), 1))
    memb = jnp.dot(onehot.astype(jnp.float32), mtab_ref[...],
                   preferred_element_type=jnp.float32)

    h = (jnp.dot(upc_ref[...], W2a_ref[...], preferred_element_type=jnp.float32)
         + jnp.dot(store_ref[...], W2b_ref[...], preferred_element_type=jnp.float32)
         + jnp.dot(memb, W2m_ref[...], preferred_element_type=jnp.float32)
         + jnp.dot(c, W2d_ref[...], preferred_element_type=jnp.float32)
         + b2_ref[...])
    h = h * jax.nn.sigmoid(h)
    h = h * lax.rsqrt(jnp.mean(h * h, axis=-1, keepdims=True) + eps) * g2_ref[...]

    out_ref[...] = (jnp.dot(h, W3_ref[...], preferred_element_type=jnp.float32)
                    + b3_ref[...])


def _full(shape):
    return pl.BlockSpec(shape, lambda i: (0,) * len(shape))


def _tc_mlp(month_ids3, continuous_feats, upc_g, store_g, mtab_pad,
            W1, b1, g1, W2a, W2b, W2m, W2d, b2, g2, W3, b3):
    return pl.pallas_call(
        _tc_body,
        grid=(_NB,),
        in_specs=[
            pl.BlockSpec((1, 1, _TILE), lambda i: (i, 0, 0)),
            pl.BlockSpec((_TILE, N_CONT), lambda i: (i, 0)),
            pl.BlockSpec((_TILE, D_UPC), lambda i: (i, 0)),
            pl.BlockSpec((_TILE, D_STORE), lambda i: (i, 0)),
            _full((16, 6)),
            _full((N_CONT, 32)),
            _full((1, 32)),
            _full((1, 32)),
            _full((D_UPC, 128)),
            _full((D_STORE, 128)),
            _full((6, 128)),
            _full((32, 128)),
            _full((1, 128)),
            _full((1, 128)),
            _full((128, D_MODEL)),
            _full((1, D_MODEL)),
        ],
        out_specs=pl.BlockSpec((_TILE, D_MODEL), lambda i: (i, 0)),
        out_shape=jax.ShapeDtypeStruct((B, D_MODEL), jnp.float32),
    )(month_ids3, continuous_feats, upc_g, store_g, mtab_pad,
      W1, b1, g1, W2a, W2b, W2m, W2d, b2, g2, W3, b3)


def kernel(upc_ids, store_ids, continuous_feats, month_ids,
           upc_table, store_table, month_table,
           W1, b1, g1, W2, b2, g2, W3, b3):
    upc_ids = upc_ids.astype(jnp.int32)
    store_ids = store_ids.astype(jnp.int32)
    month_ids3 = month_ids.astype(jnp.int32).reshape(_NB, 1, _TILE)

    upc_g, store_g = _sc_gather(upc_ids, store_ids, upc_table, store_table)

    W2a = W2[0:32]
    W2b = W2[32:48]
    W2m = W2[64:70]
    W2d = W2[70:102]
    mtab_pad = jnp.zeros((16, 6), jnp.float32).at[:12].set(month_table)

    return _tc_mlp(month_ids3, continuous_feats, upc_g, store_g, mtab_pad,
                   W1, b1.reshape(1, -1), g1.reshape(1, -1),
                   W2a, W2b, W2m, W2d, b2.reshape(1, -1), g2.reshape(1, -1),
                   W3, b3.reshape(1, -1))
